# baseline (speedup 1.0000x reference)
import jax
import jax.numpy as jnp
from jax.experimental import pallas as pl
from jax.experimental.pallas import tpu as pltpu

F = 81920
B = 1024
M = 4
BF = 1024
NSTEPS = F // BF


def _nnue_kernel(wf_ref, bf_ref, ftwT_ref, ftb_ref, turn_ref, score_ref,
                 result_ref, l1wT_ref, l1b_ref, l2wT_ref, l2b_ref,
                 out_ref, wacc, bacc):
    i = pl.program_id(0)

    @pl.when(i == 0)
    def _init():
        wacc[...] = jnp.zeros_like(wacc)
        bacc[...] = jnp.zeros_like(bacc)

    ftwT = ftwT_ref[...]
    wacc[...] += jnp.dot(wf_ref[...], ftwT, preferred_element_type=jnp.float32)
    bacc[...] += jnp.dot(bf_ref[...], ftwT, preferred_element_type=jnp.float32)

    @pl.when(i == NSTEPS - 1)
    def _epilogue():
        ftb = ftb_ref[...]
        w = wacc[...] + ftb
        b = bacc[...] + ftb
        turn = turn_ref[...]
        acc_wb = jnp.concatenate([w, b], axis=1)
        acc_bw = jnp.concatenate([b, w], axis=1)
        accumulator = turn * acc_wb + (1.0 - turn) * acc_bw
        l1_x = jnp.clip(accumulator, 0.0, 1.0)
        l2_in = jnp.dot(l1_x, l1wT_ref[...],
                        preferred_element_type=jnp.float32) + l1b_ref[...]
        l2_x = jnp.clip(l2_in, 0.0, 1.0)
        model_result = jnp.dot(l2_x, l2wT_ref[...],
                               preferred_element_type=jnp.float32) + l2b_ref[...]
        wdl_m = jax.nn.sigmoid(model_result / 400.0)
        wdl_t = jax.nn.sigmoid(score_ref[...] / 400.0)
        loss = 0.5 * (wdl_m - wdl_t) ** 2 + 0.5 * (wdl_m - result_ref[...]) ** 2
        out_ref[...] = loss


def kernel(white_features, black_features, turn, score, result,
           ft_w, ft_b, l1_w, l1_b, l2_w, l2_b):
    return pl.pallas_call(
        _nnue_kernel,
        grid=(NSTEPS,),
        in_specs=[
            pl.BlockSpec((B, BF), lambda i: (0, i)),
            pl.BlockSpec((B, BF), lambda i: (0, i)),
            pl.BlockSpec((BF, M), lambda i: (i, 0)),
            pl.BlockSpec((1, M), lambda i: (0, 0)),
            pl.BlockSpec((B, 1), lambda i: (0, 0)),
            pl.BlockSpec((B, 1), lambda i: (0, 0)),
            pl.BlockSpec((B, 1), lambda i: (0, 0)),
            pl.BlockSpec((2 * M, 8), lambda i: (0, 0)),
            pl.BlockSpec((1, 8), lambda i: (0, 0)),
            pl.BlockSpec((8, 1), lambda i: (0, 0)),
            pl.BlockSpec((1, 1), lambda i: (0, 0)),
        ],
        out_specs=pl.BlockSpec((B, 1), lambda i: (0, 0)),
        out_shape=jax.ShapeDtypeStruct((B, 1), jnp.float32),
        scratch_shapes=[pltpu.VMEM((B, M), jnp.float32),
                        pltpu.VMEM((B, M), jnp.float32)],
    )(white_features, black_features, ft_w.T, ft_b.reshape(1, M),
      turn, score, result, l1_w.T, l1_b.reshape(1, 8),
      l2_w.T, l2_b.reshape(1, 1))


# speedup vs baseline: 1.0174x; 1.0174x over previous
import jax
import jax.numpy as jnp
from jax.experimental import pallas as pl
from jax.experimental.pallas import tpu as pltpu

F = 81920
B = 1024
M = 4
BF = 2048
NSTEPS = F // BF


def _nnue_kernel(wf_ref, bf_ref, ftwT_ref, ftb_ref, turn_ref, score_ref,
                 result_ref, l1wT_ref, l1b_ref, l2wT_ref, l2b_ref,
                 out_ref, wacc, bacc):
    i = pl.program_id(0)

    @pl.when(i == 0)
    def _init():
        wacc[...] = jnp.zeros_like(wacc)
        bacc[...] = jnp.zeros_like(bacc)

    ftwT = ftwT_ref[...]
    wacc[...] += jnp.dot(wf_ref[...], ftwT, preferred_element_type=jnp.float32)
    bacc[...] += jnp.dot(bf_ref[...], ftwT, preferred_element_type=jnp.float32)

    @pl.when(i == NSTEPS - 1)
    def _epilogue():
        ftb = ftb_ref[...]
        w = wacc[...] + ftb
        b = bacc[...] + ftb
        turn = turn_ref[...]
        acc_wb = jnp.concatenate([w, b], axis=1)
        acc_bw = jnp.concatenate([b, w], axis=1)
        accumulator = turn * acc_wb + (1.0 - turn) * acc_bw
        l1_x = jnp.clip(accumulator, 0.0, 1.0)
        l2_in = jnp.dot(l1_x, l1wT_ref[...],
                        preferred_element_type=jnp.float32) + l1b_ref[...]
        l2_x = jnp.clip(l2_in, 0.0, 1.0)
        model_result = jnp.dot(l2_x, l2wT_ref[...],
                               preferred_element_type=jnp.float32) + l2b_ref[...]
        wdl_m = jax.nn.sigmoid(model_result / 400.0)
        wdl_t = jax.nn.sigmoid(score_ref[...] / 400.0)
        loss = 0.5 * (wdl_m - wdl_t) ** 2 + 0.5 * (wdl_m - result_ref[...]) ** 2
        out_ref[...] = loss


def kernel(white_features, black_features, turn, score, result,
           ft_w, ft_b, l1_w, l1_b, l2_w, l2_b):
    return pl.pallas_call(
        _nnue_kernel,
        grid=(NSTEPS,),
        in_specs=[
            pl.BlockSpec((B, BF), lambda i: (0, i)),
            pl.BlockSpec((B, BF), lambda i: (0, i)),
            pl.BlockSpec((BF, M), lambda i: (i, 0)),
            pl.BlockSpec((1, M), lambda i: (0, 0)),
            pl.BlockSpec((B, 1), lambda i: (0, 0)),
            pl.BlockSpec((B, 1), lambda i: (0, 0)),
            pl.BlockSpec((B, 1), lambda i: (0, 0)),
            pl.BlockSpec((2 * M, 8), lambda i: (0, 0)),
            pl.BlockSpec((1, 8), lambda i: (0, 0)),
            pl.BlockSpec((8, 1), lambda i: (0, 0)),
            pl.BlockSpec((1, 1), lambda i: (0, 0)),
        ],
        out_specs=pl.BlockSpec((B, 1), lambda i: (0, 0)),
        out_shape=jax.ShapeDtypeStruct((B, 1), jnp.float32),
        scratch_shapes=[pltpu.VMEM((B, M), jnp.float32),
                        pltpu.VMEM((B, M), jnp.float32)],
    )(white_features, black_features, ft_w.T, ft_b.reshape(1, M),
      turn, score, result, l1_w.T, l1_b.reshape(1, 8),
      l2_w.T, l2_b.reshape(1, 1))
